# baseline (speedup 1.0000x reference)
import jax
import jax.numpy as jnp
from jax.experimental import pallas as pl
from jax.experimental.pallas import tpu as pltpu

_K = 2
_NF = 2


def _dense_body(lg_ref, x_ref, W1_ref, b1_ref, W2_ref, b2_ref, o_ref):
    e = pl.program_id(0)
    f = pl.program_id(1)

    lg = lg_ref[...]
    m = jnp.max(lg, axis=1, keepdims=True)
    ex = jnp.exp(lg - m)
    p = ex / jnp.sum(ex, axis=1, keepdims=True)
    lane = jax.lax.broadcasted_iota(jnp.int32, p.shape, 1)
    onehot = lane == e
    pe = jnp.sum(jnp.where(onehot, p, 0.0), axis=1, keepdims=True)
    beats = (p > pe) | ((p == pe) & (lane < e))
    cnt = jnp.sum(beats.astype(jnp.int32), axis=1, keepdims=True)
    wgt = jnp.where(cnt < _K, pe, 0.0)

    x = x_ref[...]
    h = jnp.dot(x, W1_ref[0], preferred_element_type=jnp.float32)
    h = jax.nn.gelu(h + b1_ref[0]).astype(jnp.bfloat16)
    y = jnp.dot(h, W2_ref[0], preferred_element_type=jnp.float32)

    @pl.when((e == 0) & (f == 0))
    def _():
        o_ref[...] = jnp.zeros_like(o_ref)

    contrib = y * wgt
    @pl.when(f == 0)
    def _():
        o_ref[...] += b2_ref[0] * wgt

    o_ref[...] += contrib


@jax.jit
def kernel(hidden_states, router_logits, W1, b1, W2, b2):
    B_, S_, D_ = hidden_states.shape
    T = B_ * S_
    E_, _, F_ = W1.shape
    Fc = F_ // _NF

    x = hidden_states.reshape(T, D_).astype(jnp.bfloat16)
    W1b = W1.astype(jnp.bfloat16)
    W2b = W2.astype(jnp.bfloat16)
    b1r = b1.reshape(E_, 1, F_)
    b2r = b2.reshape(E_, 1, D_)

    out = pl.pallas_call(
        _dense_body,
        grid=(E_, _NF),
        in_specs=[
            pl.BlockSpec((T, E_), lambda e, f: (0, 0)),
            pl.BlockSpec((T, D_), lambda e, f: (0, 0)),
            pl.BlockSpec((1, D_, Fc), lambda e, f: (e, 0, f)),
            pl.BlockSpec((1, 1, Fc), lambda e, f: (e, 0, f)),
            pl.BlockSpec((1, Fc, D_), lambda e, f: (e, f, 0)),
            pl.BlockSpec((1, 1, D_), lambda e, f: (e, 0, 0)),
        ],
        out_specs=pl.BlockSpec((T, D_), lambda e, f: (0, 0)),
        out_shape=jax.ShapeDtypeStruct((T, D_), jnp.float32),
        compiler_params=pltpu.CompilerParams(
            dimension_semantics=("arbitrary", "arbitrary")
        ),
    )(router_logits, x, W1b, b1r, W2b, b2r)
    return out.reshape(B_, S_, D_)


# speedup vs baseline: 1.6335x; 1.6335x over previous
import jax
import jax.numpy as jnp
from jax.experimental import pallas as pl
from jax.experimental.pallas import tpu as pltpu

_K = 2
_BM = 128


def _routing_body(nb, lg_ref, bexp_ref, pos_ref, w0_ref, w1_ref):
    T, E = lg_ref.shape
    lg = lg_ref[...]
    m = jnp.max(lg, axis=1, keepdims=True)
    ex = jnp.exp(lg - m)
    p = ex / jnp.sum(ex, axis=1, keepdims=True)
    lane = jax.lax.broadcasted_iota(jnp.int32, p.shape, 1)

    w1v = jnp.max(p, axis=1, keepdims=True)
    i1 = jnp.min(jnp.where(p == w1v, lane, E), axis=1, keepdims=True)
    pm = jnp.where(lane == i1, -1.0, p)
    w2v = jnp.max(pm, axis=1, keepdims=True)
    i2 = jnp.min(jnp.where(pm == w2v, lane, E), axis=1, keepdims=True)

    O0 = (lane == i1).astype(jnp.float32)
    O1 = (lane == i2).astype(jnp.float32)
    O = jnp.concatenate([O0, O1], axis=0)

    counts = jnp.sum(O, axis=0, keepdims=True).astype(jnp.int32)
    pcount = ((counts + (_BM - 1)) // _BM) * _BM
    v = pcount
    sh = 1
    while sh < E:
        v = v + jnp.concatenate(
            [jnp.zeros((1, sh), jnp.int32), v[:, :E - sh]], axis=1)
        sh *= 2
    pe_end = v
    po = (pe_end - pcount).astype(jnp.float32)

    nc = (2 * T) // _BM
    sub = jax.lax.broadcasted_iota(jnp.int32, (_BM, _BM), 0)
    lan2 = jax.lax.broadcasted_iota(jnp.int32, (_BM, _BM), 1)
    ls = (lan2 < sub).astype(jnp.float32)
    carry = jnp.zeros((1, E), jnp.float32)
    for c in range(nc):
        oc = O[c * _BM:(c + 1) * _BM]
        rk = jnp.dot(ls, oc, preferred_element_type=jnp.float32) + carry
        pos_c = jnp.sum((rk + po) * oc, axis=1, keepdims=True)
        pos_ref[c * _BM:(c + 1) * _BM, :] = pos_c.astype(jnp.int32)
        carry = carry + jnp.sum(oc, axis=0, keepdims=True)

    w0_ref[...] = w1v
    w1_ref[...] = w2v

    bv = jax.lax.broadcasted_iota(jnp.int32, (nb, 1), 0) * _BM
    be = jnp.sum((bv >= pe_end).astype(jnp.int32), axis=1, keepdims=True)
    bexp_ref[...] = jnp.minimum(be, E - 1)


def _mlp_body(bexp_sref, r0_ref, r1_ref, w0_ref, w1_ref, x_ref, W1_ref,
              b1_ref, W2_ref, b2_ref, y_ref):
    b = pl.program_id(0)
    T = x_ref.shape[0]
    scol = jax.lax.broadcasted_iota(jnp.int32, (_BM, T), 0) + b * _BM
    m0 = r0_ref[...] == scol
    m1 = r1_ref[...] == scol
    g = (m0 | m1).astype(jnp.bfloat16)
    xb = jnp.dot(g, x_ref[...], preferred_element_type=jnp.float32)
    wm = jnp.where(m0, w0_ref[...], 0.0) + jnp.where(m1, w1_ref[...], 0.0)
    wslot = jnp.sum(wm, axis=1, keepdims=True)
    h = jnp.dot(xb.astype(jnp.bfloat16), W1_ref[0],
                preferred_element_type=jnp.float32) + b1_ref[0]
    h = jax.nn.gelu(h).astype(jnp.bfloat16)
    y = jnp.dot(h, W2_ref[0], preferred_element_type=jnp.float32) + b2_ref[0]
    y_ref[...] = (y * wslot).astype(jnp.bfloat16)


def _combine_body(r0_ref, r1_ref, y_ref, o_ref):
    tb = pl.program_id(0)
    R = y_ref.shape[0]
    rl = jax.lax.broadcasted_iota(jnp.int32, (_BM, R), 1)
    sel = ((rl == r0_ref[...]) | (rl == r1_ref[...])).astype(jnp.bfloat16)
    o_ref[...] = jnp.dot(sel, y_ref[...], preferred_element_type=jnp.float32)


@jax.jit
def kernel(hidden_states, router_logits, W1, b1, W2, b2):
    B_, S_, D_ = hidden_states.shape
    T = B_ * S_
    E_, _, F_ = W1.shape
    NB = (T * _K + E_ * (_BM - 1) + _BM - 1) // _BM
    R = NB * _BM
    NBT = T // _BM

    x = hidden_states.reshape(T, D_).astype(jnp.bfloat16)
    W1b = W1.astype(jnp.bfloat16)
    W2b = W2.astype(jnp.bfloat16)
    b1r = b1.reshape(E_, 1, F_)
    b2r = b2.reshape(E_, 1, D_)

    bexp, pos, w0, w1 = pl.pallas_call(
        lambda *refs: _routing_body(NB, *refs),
        out_shape=[
            jax.ShapeDtypeStruct((NB, 1), jnp.int32),
            jax.ShapeDtypeStruct((2 * T, 1), jnp.int32),
            jax.ShapeDtypeStruct((T, 1), jnp.float32),
            jax.ShapeDtypeStruct((T, 1), jnp.float32),
        ],
    )(router_logits)

    r0c, r1c = pos[:T], pos[T:]
    r0r, r1r = r0c.reshape(1, T), r1c.reshape(1, T)
    w0r, w1r = w0.reshape(1, T), w1.reshape(1, T)

    y = pl.pallas_call(
        _mlp_body,
        grid_spec=pltpu.PrefetchScalarGridSpec(
            num_scalar_prefetch=1,
            grid=(NB,),
            in_specs=[
                pl.BlockSpec((1, T), lambda b, be: (0, 0)),
                pl.BlockSpec((1, T), lambda b, be: (0, 0)),
                pl.BlockSpec((1, T), lambda b, be: (0, 0)),
                pl.BlockSpec((1, T), lambda b, be: (0, 0)),
                pl.BlockSpec((T, D_), lambda b, be: (0, 0)),
                pl.BlockSpec((1, D_, F_), lambda b, be: (be[b], 0, 0)),
                pl.BlockSpec((1, 1, F_), lambda b, be: (be[b], 0, 0)),
                pl.BlockSpec((1, F_, D_), lambda b, be: (be[b], 0, 0)),
                pl.BlockSpec((1, 1, D_), lambda b, be: (be[b], 0, 0)),
            ],
            out_specs=pl.BlockSpec((_BM, D_), lambda b, be: (b, 0)),
        ),
        out_shape=jax.ShapeDtypeStruct((R, D_), jnp.bfloat16),
        compiler_params=pltpu.CompilerParams(
            dimension_semantics=("arbitrary",)),
    )(bexp.reshape(NB), r0r, r1r, w0r, w1r, x, W1b, b1r, W2b, b2r)

    out = pl.pallas_call(
        _combine_body,
        grid=(NBT,),
        in_specs=[
            pl.BlockSpec((_BM, 1), lambda tb: (tb, 0)),
            pl.BlockSpec((_BM, 1), lambda tb: (tb, 0)),
            pl.BlockSpec((R, D_), lambda tb: (0, 0)),
        ],
        out_specs=pl.BlockSpec((_BM, D_), lambda tb: (tb, 0)),
        out_shape=jax.ShapeDtypeStruct((T, D_), jnp.float32),
        compiler_params=pltpu.CompilerParams(
            dimension_semantics=("arbitrary",)),
    )(r0c, r1c, y)
    return out.reshape(B_, S_, D_)
